# Initial kernel scaffold; baseline (speedup 1.0000x reference)
import jax
import jax.numpy as jnp
from jax.experimental import pallas as pl


def kernel(X, Wu, Wm, Wt, Wg, W_users, b_users, W_movies, b_movies, W_out, b_out):
    raise NotImplementedError("write your pallas kernel here")


# speedup vs baseline: 1.2460x; 1.2460x over previous
import functools

import jax
import jax.numpy as jnp
from jax import lax
from jax.experimental import pallas as pl
from jax.experimental.pallas import tpu as pltpu
from jax.experimental.pallas import tpu_sc as plsc

NC = 2
NS = 16
NW = NC * NS

BATCH = 16384
EMB = 64
HID = 128
BPW = BATCH // NW


def _sc_gather4(Xt, Wu, Wm, Wt, Wg):
    mesh = plsc.VectorSubcoreMesh(
        core_axis_name="c", subcore_axis_name="s", num_cores=NC, num_subcores=NS
    )
    out_type = [jax.ShapeDtypeStruct((BATCH, EMB), jnp.float32)] * 4

    @functools.partial(
        pl.kernel,
        out_type=out_type,
        mesh=mesh,
        scratch_types=[
            pltpu.VMEM((BPW,), jnp.int32),
            pltpu.VMEM((BPW, EMB), jnp.float32),
            pltpu.SemaphoreType.DMA,
        ],
        compiler_params=pltpu.CompilerParams(use_tc_tiling_on_sc=False),
    )
    def k(idx_hbm, wu, wm, wt, wg, ou, om, ot, og, idx_v, rows_v, sem):
        wid = lax.axis_index("s") * NC + lax.axis_index("c")
        base = wid * BPW
        tables = (wu, wm, wt, wg)
        outs = (ou, om, ot, og)
        for t in range(4):
            pltpu.sync_copy(idx_hbm.at[t, pl.ds(base, BPW)], idx_v)
            pltpu.async_copy(tables[t].at[idx_v], rows_v, sem).wait()
            pltpu.sync_copy(rows_v, outs[t].at[pl.ds(base, BPW)])

    return k(Xt, Wu, Wm, Wt, Wg)


def _tc_mlp(ue, me, te, ge, W_users, b_users, W_movies, b_movies, W_out_row, b_out):
    B = 2048
    grid = (BATCH // B,)

    def body(ue_r, me_r, te_r, ge_r, wu_r, bu_r, wm_r, bm_r, wo_r, bo_r, out_r):
        users = (
            jnp.dot(ue_r[...], wu_r[...], preferred_element_type=jnp.float32)
            + bu_r[...]
        )
        movies = (
            jnp.dot(te_r[...], wm_r[0:EMB], preferred_element_type=jnp.float32)
            + jnp.dot(me_r[...], wm_r[EMB : 2 * EMB], preferred_element_type=jnp.float32)
            + jnp.dot(ge_r[...], wm_r[2 * EMB : 3 * EMB], preferred_element_type=jnp.float32)
            + bm_r[...]
        )
        out_r[...] = jnp.sum(users * movies * wo_r[...], axis=1) + bo_r[0, 0]

    emb_spec = pl.BlockSpec((B, EMB), lambda i: (i, 0))
    full = pl.BlockSpec(index_map=lambda i: (0, 0))
    return pl.pallas_call(
        body,
        grid=grid,
        in_specs=[
            emb_spec,
            emb_spec,
            emb_spec,
            emb_spec,
            full,
            full,
            full,
            full,
            full,
            full,
        ],
        out_specs=pl.BlockSpec((B,), lambda i: (i,)),
        out_shape=jax.ShapeDtypeStruct((BATCH,), jnp.float32),
    )(ue, me, te, ge, W_users, b_users, W_movies, b_movies, W_out_row, b_out)


def kernel(X, Wu, Wm, Wt, Wg, W_users, b_users, W_movies, b_movies, W_out, b_out):
    Xt = X.T.astype(jnp.int32)
    ue, me, te, ge = _sc_gather4(Xt, Wu, Wm, Wt, Wg)
    return _tc_mlp(
        ue,
        me,
        te,
        ge,
        W_users,
        b_users.reshape(1, HID),
        W_movies,
        b_movies.reshape(1, HID),
        W_out.reshape(1, HID),
        b_out.reshape(1, 1),
    )


# speedup vs baseline: 3.8109x; 3.0585x over previous
import functools

import jax
import jax.numpy as jnp
from jax import lax
from jax.experimental import pallas as pl
from jax.experimental.pallas import tpu as pltpu
from jax.experimental.pallas import tpu_sc as plsc

NC = 2
NS = 16
NW = NC * NS

VOCAB = 100000
BATCH = 16384
EMB = 64
HID = 128
FEAT = 4 * EMB
FPW = FEAT // NW
CHUNK = 4096


def _sc_gather_t(XT, WuT, WtT, WmT, WgT):
    mesh = plsc.VectorSubcoreMesh(
        core_axis_name="c", subcore_axis_name="s", num_cores=NC, num_subcores=NS
    )

    @functools.partial(
        pl.kernel,
        out_type=jax.ShapeDtypeStruct((FEAT, BATCH), jnp.float32),
        mesh=mesh,
        scratch_types=[
            pltpu.VMEM((VOCAB,), jnp.float32),
            pltpu.VMEM((BATCH,), jnp.int32),
            pltpu.VMEM((CHUNK,), jnp.float32),
        ],
        compiler_params=pltpu.CompilerParams(
            use_tc_tiling_on_sc=True, needs_layout_passes=False
        ),
    )
    def k(xt_hbm, wut, wtt, wmt, wgt, et_hbm, row_v, idx_v, out_v):
        wid = lax.axis_index("s") * NC + lax.axis_index("c")
        fb = wid % 8

        plan = ((wut, 0), (wtt, 2), (wmt, 1), (wgt, 3))
        for t, (tbl, xrow) in enumerate(plan):

            @pl.when(wid // 8 == t)
            def _():
                pltpu.sync_copy(xt_hbm.at[xrow, :], idx_v)
                for j in range(FPW):
                    f = fb * FPW + j
                    pltpu.sync_copy(tbl.at[f, :], row_v)
                    for ch in range(BATCH // CHUNK):

                        @plsc.parallel_loop(0, CHUNK // 16, unroll=8)
                        def _(g):
                            iv = idx_v[pl.ds(ch * CHUNK + g * 16, 16)]
                            out_v[pl.ds(g * 16, 16)] = plsc.load_gather(row_v, [iv])

                        pltpu.sync_copy(
                            out_v, et_hbm.at[t * EMB + f, pl.ds(ch * CHUNK, CHUNK)]
                        )

    return k(XT, WuT, WtT, WmT, WgT)


def _tc_mlp_t(ET, WuT, bu, WmT, bm, wo, bo):
    CB = 2048
    grid = (BATCH // CB,)

    def body(et_r, wut_r, bu_r, wmt_r, bm_r, wo_r, bo_r, out_r):
        e = et_r[...]
        users = (
            jnp.dot(wut_r[...], e[0:EMB, :], preferred_element_type=jnp.float32)
            + bu_r[...]
        )
        movies = (
            jnp.dot(wmt_r[...], e[EMB:FEAT, :], preferred_element_type=jnp.float32)
            + bm_r[...]
        )
        out_r[...] = jnp.sum(users * movies * wo_r[...], axis=0) + bo_r[0, 0]

    full = pl.BlockSpec(index_map=lambda i: (0, 0))
    return pl.pallas_call(
        body,
        grid=grid,
        in_specs=[
            pl.BlockSpec((FEAT, CB), lambda i: (0, i)),
            full,
            full,
            full,
            full,
            full,
            full,
        ],
        out_specs=pl.BlockSpec((CB,), lambda i: (i,)),
        out_shape=jax.ShapeDtypeStruct((BATCH,), jnp.float32),
    )(ET, WuT, bu, WmT, bm, wo, bo)


def kernel(X, Wu, Wm, Wt, Wg, W_users, b_users, W_movies, b_movies, W_out, b_out):
    XT = X.T.astype(jnp.int32)
    ET = _sc_gather_t(XT, Wu.T, Wt.T, Wm.T, Wg.T)
    return _tc_mlp_t(
        ET,
        W_users.T,
        b_users.reshape(HID, 1),
        W_movies.T,
        b_movies.reshape(HID, 1),
        W_out,
        b_out.reshape(1, 1),
    )
